# baseline (speedup 1.0000x reference)
import functools
import jax
import jax.numpy as jnp
from jax import lax
from jax.experimental import pallas as pl
from jax.experimental.pallas import tpu as pltpu, tpu_sc as plsc

_N = 1000
_B = 16384
_CC = 128
_PARTS = (256, 256, 256, 232)


@functools.cache
def _make_sc_kernel():
    info = plsc.get_sparse_core_info()
    NC, NS, L = info.num_cores, info.num_subcores, info.num_lanes
    NW = NC * NS
    cols_per_w = _B // NW
    n_chunks = cols_per_w // _CC
    n_parts = len(_PARTS)
    row_lo = [sum(_PARTS[:p]) for p in range(n_parts)]
    mesh = plsc.VectorSubcoreMesh(core_axis_name="c", subcore_axis_name="s")

    @functools.partial(
        pl.kernel, mesh=mesh,
        compiler_params=pltpu.CompilerParams(
            use_tc_tiling_on_sc=True, needs_layout_passes=False),
        out_type=jax.ShapeDtypeStruct((_N, _B), jnp.float32),
        scratch_types=(
            [pltpu.VMEM((cols_per_w,), jnp.int32)]
            + [pltpu.VMEM((nr, _CC), jnp.float32) for nr in _PARTS]
            + [pltpu.SemaphoreType.DMA] * n_parts
        ),
    )
    def k(idx_hbm, out_hbm, idx_v, *bufs_sems):
        bufs = bufs_sems[:n_parts]
        sems = bufs_sems[n_parts:]
        wid = lax.axis_index("s") * NC + lax.axis_index("c")
        base = wid * cols_per_w
        pltpu.sync_copy(idx_hbm.at[pl.ds(base, cols_per_w)], idx_v)
        zeros = jnp.zeros((L,), jnp.float32)
        ones = jnp.ones((L,), jnp.float32)
        lane = lax.iota(jnp.int32, L)

        def zfill(p):
            def zrow(r, carry):
                for j in range(_CC // L):
                    bufs[p][r, pl.ds(j * L, L)] = zeros
                return carry
            lax.fori_loop(0, _PARTS[p], zrow, 0)

        def scatter(p, c, val):
            lo, nr = row_lo[p], _PARTS[p]
            for g in range(_CC // L):
                rows = idx_v[pl.ds(c * _CC + g * L, L)]
                r = jnp.clip(rows - lo, 0, nr - 1)
                m = (rows >= lo) & (rows < lo + nr)
                plsc.store_scatter(bufs[p], [r, g * L + lane], val, mask=m)

        def dma(p, c):
            return pltpu.async_copy(
                bufs[p],
                out_hbm.at[pl.ds(row_lo[p], _PARTS[p]),
                           pl.ds(base + c * _CC, _CC)],
                sems[p])

        handles = [None] * n_parts
        for c in range(n_chunks):
            for p in range(n_parts):
                if c == 0:
                    zfill(p)
                else:
                    handles[p].wait()
                    scatter(p, c - 1, zeros)
                scatter(p, c, ones)
                handles[p] = dma(p, c)
        for p in range(n_parts):
            handles[p].wait()

    return k


def kernel(inputs, indices):
    del inputs
    return _make_sc_kernel()(indices).T


# speedup vs baseline: 1.0217x; 1.0217x over previous
import functools
import jax
import jax.numpy as jnp
from jax import lax
from jax.experimental import pallas as pl
from jax.experimental.pallas import tpu as pltpu, tpu_sc as plsc

_N = 1000
_B = 16384
_CC = 128
_PARTS = (504, 496)


@functools.cache
def _make_sc_kernel():
    info = plsc.get_sparse_core_info()
    NC, NS, L = info.num_cores, info.num_subcores, info.num_lanes
    NW = NC * NS
    cols_per_w = _B // NW
    n_chunks = cols_per_w // _CC
    n_parts = len(_PARTS)
    row_lo = [sum(_PARTS[:p]) for p in range(n_parts)]
    mesh = plsc.VectorSubcoreMesh(core_axis_name="c", subcore_axis_name="s")

    @functools.partial(
        pl.kernel, mesh=mesh,
        compiler_params=pltpu.CompilerParams(
            use_tc_tiling_on_sc=True, needs_layout_passes=False,
            skip_device_barrier=True, disable_bounds_checks=True,
            disable_semaphore_checks=True),
        out_type=jax.ShapeDtypeStruct((_N, _B), jnp.float32),
        scratch_types=(
            [pltpu.VMEM((cols_per_w,), jnp.int32)]
            + [pltpu.VMEM((nr, _CC), jnp.float32) for nr in _PARTS]
            + [pltpu.SemaphoreType.DMA] * n_parts
        ),
    )
    def k(idx_hbm, out_hbm, idx_v, *bufs_sems):
        bufs = bufs_sems[:n_parts]
        sems = bufs_sems[n_parts:]
        wid = lax.axis_index("s") * NC + lax.axis_index("c")
        base = wid * cols_per_w
        pltpu.sync_copy(idx_hbm.at[pl.ds(base, cols_per_w)], idx_v)
        zeros = jnp.zeros((L,), jnp.float32)
        ones = jnp.ones((L,), jnp.float32)
        lane = lax.iota(jnp.int32, L)

        def zfill(p):
            def zrow(r, carry):
                for j in range(_CC // L):
                    bufs[p][r, pl.ds(j * L, L)] = zeros
                return carry
            lax.fori_loop(0, _PARTS[p], zrow, 0)

        def scatter(p, c, val):
            lo, nr = row_lo[p], _PARTS[p]
            for g in range(_CC // L):
                rows = idx_v[pl.ds(c * _CC + g * L, L)]
                r = jnp.clip(rows - lo, 0, nr - 1)
                m = (rows >= lo) & (rows < lo + nr)
                plsc.store_scatter(bufs[p], [r, g * L + lane], val, mask=m)

        def dma(p, c):
            return pltpu.async_copy(
                bufs[p],
                out_hbm.at[pl.ds(row_lo[p], _PARTS[p]),
                           pl.ds(base + c * _CC, _CC)],
                sems[p])

        handles = [None] * n_parts
        for c in range(n_chunks):
            for p in range(n_parts):
                if c == 0:
                    zfill(p)
                else:
                    handles[p].wait()
                    scatter(p, c - 1, zeros)
                scatter(p, c, ones)
                handles[p] = dma(p, c)
        for p in range(n_parts):
            handles[p].wait()

    return k


def kernel(inputs, indices):
    del inputs
    return _make_sc_kernel()(indices).T


# speedup vs baseline: 1.0250x; 1.0033x over previous
import functools
import jax
import jax.numpy as jnp
from jax import lax
from jax.experimental import pallas as pl
from jax.experimental.pallas import tpu as pltpu, tpu_sc as plsc

_N = 1000
_B = 16384
_CC = 128
_PARTS = (504, 496)


@functools.cache
def _make_sc_kernel():
    info = plsc.get_sparse_core_info()
    NC, NS, L = info.num_cores, info.num_subcores, info.num_lanes
    NW = NC * NS
    cols_per_w = _B // NW
    n_chunks = cols_per_w // _CC
    n_parts = len(_PARTS)
    row_lo = [sum(_PARTS[:p]) for p in range(n_parts)]
    mesh = plsc.VectorSubcoreMesh(core_axis_name="c", subcore_axis_name="s")

    @functools.partial(
        pl.kernel, mesh=mesh,
        compiler_params=pltpu.CompilerParams(
            use_tc_tiling_on_sc=True, needs_layout_passes=False),
        out_type=jax.ShapeDtypeStruct((_N, _B), jnp.float32),
        scratch_types=(
            [pltpu.VMEM((cols_per_w,), jnp.int32)]
            + [pltpu.VMEM((nr, _CC), jnp.float32) for nr in _PARTS]
            + [pltpu.SemaphoreType.DMA] * n_parts
        ),
    )
    def k(idx_hbm, out_hbm, idx_v, *bufs_sems):
        bufs = bufs_sems[:n_parts]
        sems = bufs_sems[n_parts:]
        wid = lax.axis_index("s") * NC + lax.axis_index("c")
        base = wid * cols_per_w
        pltpu.sync_copy(idx_hbm.at[pl.ds(base, cols_per_w)], idx_v)
        zeros = jnp.zeros((L,), jnp.float32)
        ones = jnp.ones((L,), jnp.float32)
        lane = lax.iota(jnp.int32, L)

        def zfill(p):
            def zrow(r, carry):
                for j in range(_CC // L):
                    bufs[p][r, pl.ds(j * L, L)] = zeros
                return carry
            lax.fori_loop(0, _PARTS[p], zrow, 0)

        def scatter(p, c, val):
            lo, nr = row_lo[p], _PARTS[p]
            for g in range(_CC // L):
                rows = idx_v[pl.ds(c * _CC + g * L, L)]
                r = jnp.clip(rows - lo, 0, nr - 1)
                m = (rows >= lo) & (rows < lo + nr)
                plsc.store_scatter(bufs[p], [r, g * L + lane], val, mask=m)

        def dma(p, c):
            return pltpu.async_copy(
                bufs[p],
                out_hbm.at[pl.ds(row_lo[p], _PARTS[p]),
                           pl.ds(base + c * _CC, _CC)],
                sems[p])

        handles = [None] * n_parts
        for c in range(n_chunks):
            for p in range(n_parts):
                if c == 0:
                    zfill(p)
                else:
                    handles[p].wait()
                    scatter(p, c - 1, zeros)
                scatter(p, c, ones)
                handles[p] = dma(p, c)
        for p in range(n_parts):
            handles[p].wait()

    return k


def kernel(inputs, indices):
    del inputs
    return _make_sc_kernel()(indices).T


# speedup vs baseline: 1.0337x; 1.0085x over previous
import functools
import jax
import jax.numpy as jnp
from jax import lax
from jax.experimental import pallas as pl
from jax.experimental.pallas import tpu as pltpu, tpu_sc as plsc

_N = 1000
_B = 16384
_CC = 128
_PARTS = (504, 496)


@functools.cache
def _make_sc_kernel():
    info = plsc.get_sparse_core_info()
    NC, NS, L = info.num_cores, info.num_subcores, info.num_lanes
    NW = NC * NS
    cols_per_w = _B // NW
    n_chunks = cols_per_w // _CC
    n_parts = len(_PARTS)
    row_lo = [sum(_PARTS[:p]) for p in range(n_parts)]
    mesh = plsc.VectorSubcoreMesh(core_axis_name="c", subcore_axis_name="s")

    @functools.partial(
        pl.kernel, mesh=mesh,
        compiler_params=pltpu.CompilerParams(
            use_tc_tiling_on_sc=True, needs_layout_passes=False),
        out_type=jax.ShapeDtypeStruct((_N, _B), jnp.float32),
        scratch_types=(
            [pltpu.VMEM((cols_per_w,), jnp.int32)]
            + [pltpu.VMEM((nr, _CC), jnp.float32) for nr in _PARTS]
            + [pltpu.SemaphoreType.DMA] * n_parts
        ),
    )
    def k(idx_hbm, out_hbm, idx_v, *bufs_sems):
        bufs = bufs_sems[:n_parts]
        sems = bufs_sems[n_parts:]
        wid = lax.axis_index("s") * NC + lax.axis_index("c")
        base = wid * cols_per_w
        idx_h = pltpu.async_copy(
            idx_hbm.at[pl.ds(base, cols_per_w)], idx_v, sems[0])
        zeros = jnp.zeros((L,), jnp.float32)
        ones = jnp.ones((L,), jnp.float32)
        lane = lax.iota(jnp.int32, L)

        def zfill(p):
            def zrow(r, carry):
                for j in range(_CC // L):
                    bufs[p][r, pl.ds(j * L, L)] = zeros
                return carry
            lax.fori_loop(0, _PARTS[p], zrow, 0)

        def scatter(p, c, val):
            lo, nr = row_lo[p], _PARTS[p]
            for g in range(_CC // L):
                rows = idx_v[pl.ds(c * _CC + g * L, L)]
                r = jnp.clip(rows - lo, 0, nr - 1)
                m = (rows >= lo) & (rows < lo + nr)
                plsc.store_scatter(bufs[p], [r, g * L + lane], val, mask=m)

        def dma(p, c):
            return pltpu.async_copy(
                bufs[p],
                out_hbm.at[pl.ds(row_lo[p], _PARTS[p]),
                           pl.ds(base + c * _CC, _CC)],
                sems[p])

        handles = [None] * n_parts
        for c in range(n_chunks):
            for p in range(n_parts):
                if c == 0:
                    zfill(p)
                    if p == 0:
                        idx_h.wait()
                else:
                    handles[p].wait()
                    scatter(p, c - 1, zeros)
                scatter(p, c, ones)
                handles[p] = dma(p, c)
        for p in range(n_parts):
            handles[p].wait()

    return k


def kernel(inputs, indices):
    del inputs
    return _make_sc_kernel()(indices).T


# speedup vs baseline: 1.0484x; 1.0142x over previous
import functools
import jax
import jax.numpy as jnp
from jax import lax
from jax.experimental import pallas as pl
from jax.experimental.pallas import tpu as pltpu, tpu_sc as plsc

_N = 1000
_B = 16384
_CC = 128
_PARTS = (504, 496)


@functools.cache
def _make_sc_kernel():
    info = plsc.get_sparse_core_info()
    NC, NS, L = info.num_cores, info.num_subcores, info.num_lanes
    NW = NC * NS
    cols_per_w = _B // NW
    n_chunks = cols_per_w // _CC
    n_parts = len(_PARTS)
    row_lo = [sum(_PARTS[:p]) for p in range(n_parts)]
    mesh = plsc.VectorSubcoreMesh(core_axis_name="c", subcore_axis_name="s")

    @functools.partial(
        pl.kernel, mesh=mesh,
        compiler_params=pltpu.CompilerParams(
            use_tc_tiling_on_sc=True, needs_layout_passes=False),
        out_type=jax.ShapeDtypeStruct((_N, _B), jnp.float32),
        scratch_types=(
            [pltpu.VMEM((cols_per_w,), jnp.int32)]
            + [pltpu.VMEM((nr, _CC), jnp.float32) for nr in _PARTS]
            + [pltpu.SemaphoreType.DMA] * n_parts
        ),
    )
    def k(idx_hbm, out_hbm, idx_v, *bufs_sems):
        bufs = bufs_sems[:n_parts]
        sems = bufs_sems[n_parts:]
        wid = lax.axis_index("s") * NC + lax.axis_index("c")
        base = wid * cols_per_w
        idx_h = pltpu.async_copy(
            idx_hbm.at[pl.ds(base, cols_per_w)], idx_v, sems[0])
        zeros = jnp.zeros((L,), jnp.float32)
        ones = jnp.ones((L,), jnp.float32)
        lane = lax.iota(jnp.int32, L)

        def zfill(p, r0, r1):
            def zrow(r, carry):
                for j in range(_CC // L):
                    bufs[p][r, pl.ds(j * L, L)] = zeros
                return carry
            lax.fori_loop(r0, r1, zrow, 0)

        def scatter(p, c, val, r0=0, r1=None):
            lo, nr = row_lo[p], _PARTS[p]
            if r1 is None:
                r1 = nr
            for g in range(_CC // L):
                rows = idx_v[pl.ds(c * _CC + g * L, L)]
                r = jnp.clip(rows - lo, 0, nr - 1)
                m = (rows >= lo + r0) & (rows < lo + r1)
                plsc.store_scatter(bufs[p], [r, g * L + lane], val, mask=m)

        def dma(p, c, r0=0, r1=None):
            if r1 is None:
                r1 = _PARTS[p]
            return pltpu.async_copy(
                bufs[p].at[pl.ds(r0, r1 - r0)],
                out_hbm.at[pl.ds(row_lo[p] + r0, r1 - r0),
                           pl.ds(base + c * _CC, _CC)],
                sems[p])

        _SUB = 256
        first_handles = []
        for p in range(n_parts):
            zfill(p, 0, _SUB)
            if p == 0:
                idx_h.wait()
            scatter(p, 0, ones, 0, _SUB)
            first_handles.append(dma(p, 0, 0, _SUB))
            zfill(p, _SUB, _PARTS[p])
            scatter(p, 0, ones, _SUB, _PARTS[p])
            first_handles.append(dma(p, 0, _SUB, _PARTS[p]))
        handles = [None] * n_parts
        for c in range(1, n_chunks):
            for p in range(n_parts):
                if c == 1:
                    first_handles[2 * p].wait()
                    first_handles[2 * p + 1].wait()
                else:
                    handles[p].wait()
                scatter(p, c - 1, zeros)
                scatter(p, c, ones)
                handles[p] = dma(p, c)
        for p in range(n_parts):
            handles[p].wait()

    return k


def kernel(inputs, indices):
    del inputs
    return _make_sc_kernel()(indices).T
